# baseline (speedup 1.0000x reference)
import jax
import jax.numpy as jnp
from jax.experimental import pallas as pl
from jax.experimental.pallas import tpu as pltpu

N = 512
D = 128
EDGE_DIM = 16
JB = 32
STEPS = N // JB
THR = 0.5


def _body(xj_ref, x_ref, WT_ref, Mb_ref, ef_ref,
          w1aTd_ref, w1bTd_ref, w1eTd_ref, b1d_ref, w2Td_ref, b2d_ref,
          w1aTb_ref, w1bTb_ref, b1b_ref, w2Tb_ref, b2b_ref,
          nw1T_ref, nb1_ref, nw2T_ref, nb2_ref, lng_ref, lnb_ref,
          out_ref,
          Bd_ref, Bb_ref, Sd_ref, Sb_ref, swd_ref, swb_ref):
    jb = pl.program_id(0)

    @pl.when(jb == 0)
    def _init():
        x = x_ref[...]
        Bd_ref[...] = x @ w1bTd_ref[...] + b1d_ref[...]
        Bb_ref[...] = x @ w1bTb_ref[...] + b1b_ref[...]
        Sd_ref[...] = jnp.zeros((N, D), jnp.float32)
        Sb_ref[...] = jnp.zeros((N, D), jnp.float32)
        swd_ref[...] = jnp.zeros((N, 1), jnp.float32)
        swb_ref[...] = jnp.zeros((N, 1), jnp.float32)

    xj = xj_ref[...]
    E = ef_ref[...] @ w1eTd_ref[...]
    Ad = xj @ w1aTd_ref[...]
    Ab = xj @ w1aTb_ref[...]

    aWT = jnp.abs(WT_ref[0])
    wtT_d = jnp.where(aWT > THR, aWT, 0.0)
    aMb = jnp.abs(Mb_ref[0])
    rid = jax.lax.broadcasted_iota(jnp.int32, (N, JB), 0)
    cid = jb * JB + jax.lax.broadcasted_iota(jnp.int32, (N, JB), 1)
    wtT_b = jnp.where((aMb > THR) & (rid != cid), aMb, 0.0)
    swd_ref[...] += jnp.sum(wtT_d, axis=1, keepdims=True)
    swb_ref[...] += jnp.sum(wtT_b, axis=1, keepdims=True)

    G = 2

    Bb = Bb_ref[...]
    acc_b = Sb_ref[...]
    for g in range(0, JB, G):
        acc_b += sum(
            wtT_b[:, j:j + 1] * jnp.maximum(Ab[j:j + 1, :] + Bb, 0.0)
            for j in range(g, g + G))
    Sb_ref[...] = acc_b

    Bd = Bd_ref[...]
    acc_d = Sd_ref[...]
    for g in range(0, JB, G):
        acc_d += sum(
            wtT_d[:, j:j + 1]
            * jnp.maximum(Ad[j:j + 1, :] + Bd + E[j * N:(j + 1) * N, :], 0.0)
            for j in range(g, g + G))
    Sd_ref[...] = acc_d

    @pl.when(jb == STEPS - 1)
    def _fin():
        x = x_ref[...]
        hd = Sd_ref[...] @ w2Td_ref[...] + swd_ref[...] * b2d_ref[...]
        hb = Sb_ref[...] @ w2Tb_ref[...] + swb_ref[...] * b2b_ref[...]
        hs = (jnp.maximum(x @ nw1T_ref[...] + nb1_ref[...], 0.0)
              @ nw2T_ref[...] + nb2_ref[...])
        h = hs + hd + hb
        mean = jnp.mean(h, axis=1, keepdims=True)
        c = h - mean
        var = jnp.mean(c * c, axis=1, keepdims=True)
        out_ref[...] = (c * jax.lax.rsqrt(var + 1e-5) * lng_ref[...]
                        + lnb_ref[...])


def kernel(node_features, W, M, edge_features, node_w1, node_b1, node_w2,
           node_b2, dir_w1, dir_b1, dir_w2, dir_b2, bi_w1, bi_b1, bi_w2,
           bi_b2, ln_g, ln_b):
    x = node_features
    WT = W.T.reshape(N, STEPS, JB).transpose(1, 0, 2)
    Mc = M.reshape(N, STEPS, JB).transpose(1, 0, 2)
    ef2 = edge_features.reshape(N * N, EDGE_DIM)
    r1 = lambda v: v.reshape(1, D)
    full = lambda shape: pl.BlockSpec(shape, lambda j: (0, 0))
    grid_spec = pltpu.PrefetchScalarGridSpec(
        num_scalar_prefetch=0,
        grid=(STEPS,),
        in_specs=[
            pl.BlockSpec((JB, D), lambda j: (j, 0)),
            full((N, D)),
            pl.BlockSpec((1, N, JB), lambda j: (j, 0, 0)),
            pl.BlockSpec((1, N, JB), lambda j: (j, 0, 0)),
            pl.BlockSpec((JB * N, EDGE_DIM), lambda j: (j, 0)),
            full((D, D)), full((D, D)), full((EDGE_DIM, D)),
            full((1, D)), full((D, D)), full((1, D)),
            full((D, D)), full((D, D)),
            full((1, D)), full((D, D)), full((1, D)),
            full((D, D)), full((1, D)), full((D, D)), full((1, D)),
            full((1, D)), full((1, D)),
        ],
        out_specs=pl.BlockSpec((N, D), lambda j: (0, 0)),
        scratch_shapes=[pltpu.VMEM((N, D), jnp.float32)] * 4
        + [pltpu.VMEM((N, 1), jnp.float32)] * 2,
    )
    out = pl.pallas_call(
        _body,
        grid_spec=grid_spec,
        out_shape=jax.ShapeDtypeStruct((N, D), jnp.float32),
    )(x, x, WT, Mc, ef2,
      dir_w1[:, :D].T, dir_w1[:, D:2 * D].T, dir_w1[:, 2 * D:].T,
      r1(dir_b1), dir_w2.T, r1(dir_b2),
      bi_w1[:, :D].T, bi_w1[:, D:].T, r1(bi_b1), bi_w2.T, r1(bi_b2),
      node_w1.T, r1(node_b1), node_w2.T, r1(node_b2),
      r1(ln_g), r1(ln_b))
    return out
